# baseline (speedup 1.0000x reference)
import functools

import jax
import jax.numpy as jnp
from jax import lax
from jax.experimental import pallas as pl
from jax.experimental.pallas import tpu as pltpu
from jax.experimental.pallas import tpu_sc as plsc

B, L = 4096, 200
GLOVE, FEAT = 128, 64
D = GLOVE + FEAT
VOCAB = 100000

NC, NS = 2, 16
NW = NC * NS
B_PER_W = B // NW
GA, GB = 104, 96

_mesh = plsc.VectorSubcoreMesh(
    core_axis_name="c", subcore_axis_name="s", num_cores=NC, num_subcores=NS
)


@functools.partial(
    pl.kernel,
    out_type=jax.ShapeDtypeStruct((B, L, D), jnp.float32),
    mesh=_mesh,
    scratch_types=[
        pltpu.VMEM((L,), jnp.int32),
        pltpu.VMEM((L, D), jnp.float32),
        pltpu.VMEM((L, GLOVE), jnp.float32),
        pltpu.SemaphoreType.DMA,
    ],
)
def _sc_gather(x_hbm, em_hbm, ofp_hbm, out_hbm, idx_v, comb_v, of_v, sem):
    wid = lax.axis_index("s") * NC + lax.axis_index("c")
    b_base = wid * B_PER_W

    @pl.loop(0, B_PER_W)
    def _row(i):
        b = b_base + i
        pltpu.sync_copy(x_hbm.at[b], idx_v)
        descs = [
            pltpu.async_copy(
                em_hbm.at[idx_v.at[pl.ds(0, GA)]],
                comb_v.at[pl.ds(0, GA), pl.ds(0, GLOVE)],
                sem,
            ),
            pltpu.async_copy(
                em_hbm.at[idx_v.at[pl.ds(GA, GB)]],
                comb_v.at[pl.ds(GA, GB), pl.ds(0, GLOVE)],
                sem,
            ),
            pltpu.async_copy(
                ofp_hbm.at[idx_v.at[pl.ds(0, GA)]], of_v.at[pl.ds(0, GA)], sem
            ),
            pltpu.async_copy(
                ofp_hbm.at[idx_v.at[pl.ds(GA, GB)]], of_v.at[pl.ds(GA, GB)], sem
            ),
        ]
        for d in descs:
            d.wait()

        @pl.loop(0, L)
        def _asm(r):
            for c in range(FEAT // 16):
                comb_v[r, pl.ds(GLOVE + c * 16, 16)] = of_v[r, pl.ds(c * 16, 16)]

        pltpu.sync_copy(comb_v, out_hbm.at[b])


def _pad_body(of_ref, o_ref):
    o_ref[:, 0:FEAT] = of_ref[...]
    o_ref[:, FEAT:GLOVE] = jnp.zeros((of_ref.shape[0], GLOVE - FEAT), jnp.float32)


_PAD_ROWS = 2000
_pad_call = pl.pallas_call(
    _pad_body,
    grid=(VOCAB // _PAD_ROWS,),
    in_specs=[pl.BlockSpec((_PAD_ROWS, FEAT), lambda i: (i, 0))],
    out_specs=pl.BlockSpec((_PAD_ROWS, GLOVE), lambda i: (i, 0)),
    out_shape=jax.ShapeDtypeStruct((VOCAB, GLOVE), jnp.float32),
)


def _mask_body(x_ref, o_ref):
    o_ref[...] = x_ref[...] > 0


_mask_call = pl.pallas_call(
    _mask_body,
    out_shape=jax.ShapeDtypeStruct((B, L), jnp.bool_),
)


def kernel(x, em_weight, of_weight):
    of_p = _pad_call(of_weight)
    out = _sc_gather(x, em_weight, of_p)
    mask = _mask_call(x)
    return out, mask


# speedup vs baseline: 1.5514x; 1.5514x over previous
import functools

import jax
import jax.numpy as jnp
from jax import lax
from jax.experimental import pallas as pl
from jax.experimental.pallas import tpu as pltpu
from jax.experimental.pallas import tpu_sc as plsc

B, L = 4096, 200
GLOVE, FEAT = 128, 64
D = GLOVE + FEAT
DP = 256
VOCAB = 100000

NC, NS = 2, 16
NW = NC * NS
NSLICE = 8
BS = B // NSLICE
B_PER_W = BS // NW
GA, GB = 104, 96

_mesh = plsc.VectorSubcoreMesh(
    core_axis_name="c", subcore_axis_name="s", num_cores=NC, num_subcores=NS
)


def _make_sc_gather(slice_idx):
    @functools.partial(
        pl.kernel,
        out_type=jax.ShapeDtypeStruct((BS, L, DP), jnp.float32),
        mesh=_mesh,
        scratch_types=[
            pltpu.VMEM((L,), jnp.int32),
            pltpu.VMEM((L,), jnp.int32),
            pltpu.VMEM((2, L, DP), jnp.float32),
            pltpu.SemaphoreType.DMA,
            pltpu.SemaphoreType.DMA,
            pltpu.SemaphoreType.DMA,
            pltpu.SemaphoreType.DMA,
        ],
    )
    def _sc_gather(x_hbm, tab_hbm, mid_hbm, idx_v0, idx_v1, comb_v, sg0, sg1, sw0, sw1):
        wid = lax.axis_index("s") * NC + lax.axis_index("c")
        b_src0 = slice_idx * BS + wid * B_PER_W
        idx = (idx_v0, idx_v1)
        sg = (sg0, sg1)
        sw = (sw0, sw1)

        def fire(i, par):
            pltpu.sync_copy(x_hbm.at[b_src0 + i], idx[par])
            pltpu.async_copy(
                tab_hbm.at[idx[par].at[pl.ds(0, GA)]],
                comb_v.at[par, pl.ds(0, GA)],
                sg[par],
            )
            pltpu.async_copy(
                tab_hbm.at[idx[par].at[pl.ds(GA, GB)]],
                comb_v.at[par, pl.ds(GA, GB)],
                sg[par],
            )

        def wait_gathers(par):
            pltpu.make_async_copy(
                tab_hbm.at[idx[par].at[pl.ds(0, GA)]],
                comb_v.at[par, pl.ds(0, GA)],
                sg[par],
            ).wait()
            pltpu.make_async_copy(
                tab_hbm.at[idx[par].at[pl.ds(GA, GB)]],
                comb_v.at[par, pl.ds(GA, GB)],
                sg[par],
            ).wait()

        def write(i, par):
            pltpu.async_copy(
                comb_v.at[par], mid_hbm.at[wid * B_PER_W + i], sw[par]
            )

        def wait_write(i, par):
            pltpu.make_async_copy(
                comb_v.at[par], mid_hbm.at[wid * B_PER_W + i], sw[par]
            ).wait()

        fire(0, 0)

        @pl.loop(0, B_PER_W // 2)
        def _blk(j):
            i0 = j * 2
            i1 = i0 + 1

            @pl.when(j >= 1)
            def _():
                wait_write(i1 - 2, 1)

            fire(i1, 1)
            wait_gathers(0)
            write(i0, 0)

            @pl.when(i1 + 1 < B_PER_W)
            def _():
                wait_write(i0, 0)
                fire(i1 + 1, 0)

            wait_gathers(1)
            write(i1, 1)

        wait_write(B_PER_W - 2, 0)
        wait_write(B_PER_W - 1, 1)

    return _sc_gather


_LB = 8


def _trans_body(mid_ref, o_ref):
    for l in range(_LB):
        t = jnp.transpose(mid_ref[:, l, :], (1, 0))
        o_ref[l] = t[0:D, :]


def _trans_body_carry(mid_ref, carry_ref, o_ref):
    del carry_ref
    _trans_body(mid_ref, o_ref)


def _make_transpose(slice_idx, with_carry):
    out_spec = pl.BlockSpec((_LB, D, BS), lambda l: (l, 0, slice_idx))
    in_specs = [pl.BlockSpec((BS, _LB, DP), lambda l: (0, l, 0))]
    kwargs = {}
    body = _trans_body
    if with_carry:
        in_specs.append(pl.BlockSpec(memory_space=pl.ANY))
        kwargs["input_output_aliases"] = {1: 0}
        body = _trans_body_carry
    return pl.pallas_call(
        body,
        grid=(L // _LB,),
        in_specs=in_specs,
        out_specs=out_spec,
        out_shape=jax.ShapeDtypeStruct((L, D, B), jnp.float32),
        **kwargs,
    )


def _fuse_body(em_ref, of_ref, o_ref):
    o_ref[:, 0:GLOVE] = em_ref[...]
    o_ref[:, GLOVE:D] = of_ref[...]


_FUSE_ROWS = 2000
_fuse_call = pl.pallas_call(
    _fuse_body,
    grid=(VOCAB // _FUSE_ROWS,),
    in_specs=[
        pl.BlockSpec((_FUSE_ROWS, GLOVE), lambda i: (i, 0)),
        pl.BlockSpec((_FUSE_ROWS, FEAT), lambda i: (i, 0)),
    ],
    out_specs=pl.BlockSpec((_FUSE_ROWS, DP), lambda i: (i, 0)),
    out_shape=jax.ShapeDtypeStruct((VOCAB, DP), jnp.float32),
)


def _mask_body(x_ref, o_ref):
    o_ref[...] = x_ref[...] > 0


_mask_call = pl.pallas_call(
    _mask_body,
    out_shape=jax.ShapeDtypeStruct((B, L), jnp.bool_),
)


def kernel(x, em_weight, of_weight):
    tab = _fuse_call(em_weight, of_weight)
    out_t = None
    for s in range(NSLICE):
        mid_s = _make_sc_gather(s)(x, tab)
        if s == 0:
            out_t = _make_transpose(s, with_carry=False)(mid_s)
        else:
            out_t = _make_transpose(s, with_carry=True)(mid_s, out_t)
    out = jnp.transpose(out_t, (2, 0, 1))
    mask = _mask_call(x)
    return out, mask


# speedup vs baseline: 1.5577x; 1.0040x over previous
import functools

import jax
import jax.numpy as jnp
from jax import lax
from jax.experimental import pallas as pl
from jax.experimental.pallas import tpu as pltpu
from jax.experimental.pallas import tpu_sc as plsc

B, L = 4096, 200
GLOVE, FEAT = 128, 64
D = GLOVE + FEAT
DP = 256
VOCAB = 100000

NC, NS = 2, 16
NW = NC * NS
NSLICE = 8
BS = B // NSLICE
B_PER_W = BS // NW
GA, GB = 104, 96

_mesh = plsc.VectorSubcoreMesh(
    core_axis_name="c", subcore_axis_name="s", num_cores=NC, num_subcores=NS
)


def _make_sc_gather(slice_idx):
    @functools.partial(
        pl.kernel,
        out_type=jax.ShapeDtypeStruct((BS, L, DP), jnp.float32),
        mesh=_mesh,
        scratch_types=[
            pltpu.VMEM((L,), jnp.int32),
            pltpu.VMEM((L,), jnp.int32),
            pltpu.VMEM((2, L, DP), jnp.float32),
            pltpu.SemaphoreType.DMA,
            pltpu.SemaphoreType.DMA,
            pltpu.SemaphoreType.DMA,
            pltpu.SemaphoreType.DMA,
            pltpu.SemaphoreType.DMA,
            pltpu.SemaphoreType.DMA,
        ],
    )
    def _sc_gather(
        x_hbm, tab_hbm, mid_hbm, idx_v0, idx_v1, comb_v,
        sg0, sg1, sw0, sw1, si0, si1,
    ):
        wid = lax.axis_index("s") * NC + lax.axis_index("c")
        b_src0 = slice_idx * BS + wid * B_PER_W
        idx = (idx_v0, idx_v1)
        sg = (sg0, sg1)
        sw = (sw0, sw1)
        si = (si0, si1)

        def fire_idx(i, par):
            pltpu.async_copy(x_hbm.at[b_src0 + i], idx[par], si[par])

        def fire_gath(i, par):
            pltpu.make_async_copy(x_hbm.at[b_src0 + i], idx[par], si[par]).wait()
            pltpu.async_copy(
                tab_hbm.at[idx[par].at[pl.ds(0, GA)]],
                comb_v.at[par, pl.ds(0, GA)],
                sg[par],
            )
            pltpu.async_copy(
                tab_hbm.at[idx[par].at[pl.ds(GA, GB)]],
                comb_v.at[par, pl.ds(GA, GB)],
                sg[par],
            )

        def wait_gathers(par):
            pltpu.make_async_copy(
                tab_hbm.at[idx[par].at[pl.ds(0, GA)]],
                comb_v.at[par, pl.ds(0, GA)],
                sg[par],
            ).wait()
            pltpu.make_async_copy(
                tab_hbm.at[idx[par].at[pl.ds(GA, GB)]],
                comb_v.at[par, pl.ds(GA, GB)],
                sg[par],
            ).wait()

        def write(i, par):
            pltpu.async_copy(
                comb_v.at[par], mid_hbm.at[wid * B_PER_W + i], sw[par]
            )

        def wait_write(i, par):
            pltpu.make_async_copy(
                comb_v.at[par], mid_hbm.at[wid * B_PER_W + i], sw[par]
            ).wait()

        fire_idx(0, 0)
        fire_gath(0, 0)
        fire_idx(1, 1)

        @pl.loop(0, B_PER_W // 2)
        def _blk(j):
            i0 = j * 2
            i1 = i0 + 1

            @pl.when(j >= 1)
            def _():
                wait_write(i1 - 2, 1)

            fire_gath(i1, 1)
            wait_gathers(0)

            @pl.when(i1 + 1 < B_PER_W)
            def _():
                fire_idx(i1 + 1, 0)

            write(i0, 0)

            @pl.when(i1 + 1 < B_PER_W)
            def _():
                wait_write(i0, 0)
                fire_gath(i1 + 1, 0)

            wait_gathers(1)

            @pl.when(i1 + 2 < B_PER_W)
            def _():
                fire_idx(i1 + 2, 1)

            write(i1, 1)

        wait_write(B_PER_W - 2, 0)
        wait_write(B_PER_W - 1, 1)

    return _sc_gather


_LB = 8


def _trans_body(mid_ref, o_ref):
    for l in range(_LB):
        t = jnp.transpose(mid_ref[:, l, :], (1, 0))
        o_ref[l] = t[0:D, :]


def _trans_body_carry(mid_ref, carry_ref, o_ref):
    del carry_ref
    _trans_body(mid_ref, o_ref)


def _make_transpose(slice_idx, with_carry):
    out_spec = pl.BlockSpec((_LB, D, BS), lambda l: (l, 0, slice_idx))
    in_specs = [pl.BlockSpec((BS, _LB, DP), lambda l: (0, l, 0))]
    kwargs = {}
    body = _trans_body
    if with_carry:
        in_specs.append(pl.BlockSpec(memory_space=pl.ANY))
        kwargs["input_output_aliases"] = {1: 0}
        body = _trans_body_carry
    return pl.pallas_call(
        body,
        grid=(L // _LB,),
        in_specs=in_specs,
        out_specs=out_spec,
        out_shape=jax.ShapeDtypeStruct((L, D, B), jnp.float32),
        **kwargs,
    )


def _fuse_body(em_ref, of_ref, o_ref):
    o_ref[:, 0:GLOVE] = em_ref[...]
    o_ref[:, GLOVE:D] = of_ref[...]


_FUSE_ROWS = 2000
_fuse_call = pl.pallas_call(
    _fuse_body,
    grid=(VOCAB // _FUSE_ROWS,),
    in_specs=[
        pl.BlockSpec((_FUSE_ROWS, GLOVE), lambda i: (i, 0)),
        pl.BlockSpec((_FUSE_ROWS, FEAT), lambda i: (i, 0)),
    ],
    out_specs=pl.BlockSpec((_FUSE_ROWS, DP), lambda i: (i, 0)),
    out_shape=jax.ShapeDtypeStruct((VOCAB, DP), jnp.float32),
)


def _mask_body(x_ref, o_ref):
    o_ref[...] = x_ref[...] > 0


_mask_call = pl.pallas_call(
    _mask_body,
    out_shape=jax.ShapeDtypeStruct((B, L), jnp.bool_),
)


def kernel(x, em_weight, of_weight):
    tab = _fuse_call(em_weight, of_weight)
    out_t = None
    for s in range(NSLICE):
        mid_s = _make_sc_gather(s)(x, tab)
        if s == 0:
            out_t = _make_transpose(s, with_carry=False)(mid_s)
        else:
            out_t = _make_transpose(s, with_carry=True)(mid_s, out_t)
    out = jnp.transpose(out_t, (2, 0, 1))
    mask = _mask_call(x)
    return out, mask
